# Initial kernel scaffold; baseline (speedup 1.0000x reference)
import functools

import jax
import jax.numpy as jnp
from jax import lax
from jax.experimental import pallas as pl
from jax.experimental.pallas import tpu as pltpu
from jax.experimental.pallas import tpu_sc as plsc

N_EMBS = 257
EMB_DIM = 300
N_TOKENS = 4096 * 50

NUM_CORES = 2
NUM_SUBCORES = 16
NW = NUM_CORES * NUM_SUBCORES
TOK_PER_W = N_TOKENS // NW
CHUNK = 128
NCHUNKS = TOK_PER_W // CHUNK

_mesh = plsc.VectorSubcoreMesh(core_axis_name="c", subcore_axis_name="s")


@functools.partial(
    pl.kernel,
    out_type=jax.ShapeDtypeStruct((N_TOKENS, EMB_DIM), jnp.float32),
    mesh=_mesh,
    scratch_types=[
        pltpu.VMEM((TOK_PER_W,), jnp.int32),
        pltpu.VMEM((CHUNK, EMB_DIM), jnp.float32),
        pltpu.VMEM((CHUNK, EMB_DIM), jnp.float32),
        pltpu.SemaphoreType.DMA,
        pltpu.SemaphoreType.DMA,
    ],
)
def _gather_kernel(table_hbm, idx_hbm, out_hbm, idx_v, buf0, buf1, sem0, sem1):
    wid = lax.axis_index("s") * NUM_CORES + lax.axis_index("c")
    base = wid * TOK_PER_W
    pltpu.sync_copy(idx_hbm.at[pl.ds(base, TOK_PER_W)], idx_v)

    bufs = (buf0, buf1)
    sems = (sem0, sem1)

    def gather_start(i, b):
        pltpu.async_copy(
            table_hbm.at[idx_v.at[pl.ds(i * CHUNK, CHUNK)]], bufs[b], sems[b]
        )

    def drain_to_out(i, b):
        pltpu.make_async_copy(
            table_hbm.at[idx_v.at[pl.ds(i * CHUNK, CHUNK)]], bufs[b], sems[b]
        ).wait()
        pltpu.sync_copy(bufs[b], out_hbm.at[pl.ds(base + i * CHUNK, CHUNK)])

    gather_start(0, 0)

    def loop_body(i2, carry):
        i0 = i2 * 2
        gather_start(i0 + 1, 1)
        drain_to_out(i0, 0)

        @pl.when(i0 + 2 < NCHUNKS)
        def _():
            gather_start(i0 + 2, 0)

        drain_to_out(i0 + 1, 1)
        return carry

    lax.fori_loop(0, NCHUNKS // 2, loop_body, 0)


def kernel(sentence, table):
    idx = sentence.reshape(-1)
    out = _gather_kernel(table, idx)
    return out.reshape(sentence.shape[0], sentence.shape[1], EMB_DIM)


# speedup vs baseline: 1.2177x; 1.2177x over previous
import functools

import jax
import jax.numpy as jnp
from jax import lax
from jax.experimental import pallas as pl
from jax.experimental.pallas import tpu as pltpu
from jax.experimental.pallas import tpu_sc as plsc

N_EMBS = 257
EMB_DIM = 300
EMB_PAD = 304
N_TOKENS = 4096 * 50

NUM_CORES = 2
NUM_SUBCORES = 16
NW = NUM_CORES * NUM_SUBCORES
TOK_PER_W = N_TOKENS // NW
CHUNK = 80
NCHUNKS = TOK_PER_W // CHUNK

_mesh = plsc.VectorSubcoreMesh(core_axis_name="c", subcore_axis_name="s")


@functools.partial(
    pl.kernel,
    out_type=jax.ShapeDtypeStruct((N_TOKENS, EMB_DIM), jnp.float32),
    mesh=_mesh,
    scratch_types=[
        pltpu.VMEM((NCHUNKS, CHUNK), jnp.int32),
        pltpu.VMEM((CHUNK, EMB_PAD), jnp.float32),
        pltpu.VMEM((CHUNK, EMB_PAD), jnp.float32),
        pltpu.VMEM((CHUNK, EMB_DIM), jnp.float32),
        pltpu.VMEM((CHUNK, EMB_DIM), jnp.float32),
        pltpu.SemaphoreType.DMA,
        pltpu.SemaphoreType.DMA,
        pltpu.SemaphoreType.DMA,
        pltpu.SemaphoreType.DMA,
    ],
    compiler_params=pltpu.CompilerParams(use_tc_tiling_on_sc=False),
)
def _gather_kernel(table_hbm, idx_hbm, out_hbm, idx_v, bufa0, bufa1,
                   bufp0, bufp1, sga0, sga1, swb0, swb1):
    wid = lax.axis_index("s") * NUM_CORES + lax.axis_index("c")
    base = wid * TOK_PER_W
    pltpu.sync_copy(idx_hbm.at[wid], idx_v)

    bufa = (bufa0, bufa1)
    bufp = (bufp0, bufp1)
    sga = (sga0, sga1)
    swb = (swb0, swb1)

    def gather_start(i, b):
        pltpu.async_copy(table_hbm.at[idx_v.at[i]], bufa[b], sga[b])

    def gather_wait(i, b):
        pltpu.make_async_copy(table_hbm.at[idx_v.at[i]], bufa[b], sga[b]).wait()

    def wb_start(i, b):
        pltpu.async_copy(bufp[b], out_hbm.at[pl.ds(base + i * CHUNK, CHUNK)],
                         swb[b])

    def wb_wait(i, b):
        pltpu.make_async_copy(bufp[b],
                              out_hbm.at[pl.ds(base + i * CHUNK, CHUNK)],
                              swb[b]).wait()

    def repack(b):
        src = bufa[b]
        dst = bufp[b]

        def row(t, carry):
            for k in range(18):
                dst[t, pl.ds(16 * k, 16)] = src[t, pl.ds(16 * k, 16)]
            dst[t, pl.ds(EMB_DIM - 16, 16)] = src[t, pl.ds(EMB_DIM - 16, 16)]
            return carry

        lax.fori_loop(0, CHUNK, row, 0)

    gather_start(0, 0)

    def loop_body(i2, carry):
        i0 = i2 * 2
        gather_start(i0 + 1, 1)
        gather_wait(i0, 0)

        @pl.when(i0 >= 2)
        def _():
            wb_wait(i0 - 2, 0)

        repack(0)
        wb_start(i0, 0)

        @pl.when(i0 + 2 < NCHUNKS)
        def _():
            gather_start(i0 + 2, 0)

        gather_wait(i0 + 1, 1)

        @pl.when(i0 >= 2)
        def _():
            wb_wait(i0 - 1, 1)

        repack(1)
        wb_start(i0 + 1, 1)
        return carry

    lax.fori_loop(0, NCHUNKS // 2, loop_body, 0)
    wb_wait(NCHUNKS - 2, 0)
    wb_wait(NCHUNKS - 1, 1)


def kernel(sentence, table):
    idx = sentence.reshape(NW, NCHUNKS, CHUNK)
    table_p = jnp.pad(table, ((0, 0), (0, EMB_PAD - EMB_DIM)))
    out = _gather_kernel(table_p, idx)
    return out.reshape(sentence.shape[0], sentence.shape[1], EMB_DIM)


# speedup vs baseline: 1.6833x; 1.3824x over previous
import functools

import jax
import jax.numpy as jnp
from jax import lax
from jax.experimental import pallas as pl
from jax.experimental.pallas import tpu as pltpu
from jax.experimental.pallas import tpu_sc as plsc

N_EMBS = 257
EMB_DIM = 300
EMB_PAD = 304
N_SENT = 4096
SENT_LEN = 50

NUM_CORES = 2
NUM_SUBCORES = 16
NW = NUM_CORES * NUM_SUBCORES
SENT_PER_W = N_SENT // NW
CHUNK = SENT_LEN
NCHUNKS = SENT_PER_W

_mesh = plsc.VectorSubcoreMesh(core_axis_name="c", subcore_axis_name="s")


@functools.partial(
    pl.kernel,
    out_type=jax.ShapeDtypeStruct((N_SENT, SENT_LEN, EMB_DIM), jnp.float32),
    mesh=_mesh,
    scratch_types=[
        pltpu.VMEM((NCHUNKS, CHUNK), jnp.int32),
        pltpu.VMEM((CHUNK, EMB_PAD), jnp.float32),
        pltpu.VMEM((CHUNK, EMB_PAD), jnp.float32),
        pltpu.VMEM((CHUNK, EMB_DIM), jnp.float32),
        pltpu.VMEM((CHUNK, EMB_DIM), jnp.float32),
        pltpu.SemaphoreType.DMA,
        pltpu.SemaphoreType.DMA,
        pltpu.SemaphoreType.DMA,
        pltpu.SemaphoreType.DMA,
    ],
    compiler_params=pltpu.CompilerParams(use_tc_tiling_on_sc=False),
)
def _gather_kernel(table_hbm, idx_hbm, out_hbm, idx_v, bufa0, bufa1,
                   bufp0, bufp1, sga0, sga1, swb0, swb1):
    wid = lax.axis_index("s") * NUM_CORES + lax.axis_index("c")
    base = wid * NCHUNKS
    pltpu.sync_copy(idx_hbm.at[wid], idx_v)

    bufa = (bufa0, bufa1)
    bufp = (bufp0, bufp1)
    sga = (sga0, sga1)
    swb = (swb0, swb1)

    def gather_start(i, b):
        pltpu.async_copy(table_hbm.at[idx_v.at[i]], bufa[b], sga[b])

    def gather_wait(i, b):
        pltpu.make_async_copy(table_hbm.at[idx_v.at[i]], bufa[b], sga[b]).wait()

    def wb_start(i, b):
        pltpu.async_copy(bufp[b], out_hbm.at[base + i], swb[b])

    def wb_wait(i, b):
        pltpu.make_async_copy(bufp[b], out_hbm.at[base + i], swb[b]).wait()

    def repack(b):
        src = bufa[b]
        dst = bufp[b]

        def row(t, carry):
            for k in range(18):
                dst[t, pl.ds(16 * k, 16)] = src[t, pl.ds(16 * k, 16)]
            dst[t, pl.ds(EMB_DIM - 16, 16)] = src[t, pl.ds(EMB_DIM - 16, 16)]
            return carry

        lax.fori_loop(0, CHUNK, row, 0)

    gather_start(0, 0)

    def loop_body(i2, carry):
        i0 = i2 * 2
        gather_start(i0 + 1, 1)
        gather_wait(i0, 0)

        @pl.when(i0 >= 2)
        def _():
            wb_wait(i0 - 2, 0)

        repack(0)
        wb_start(i0, 0)

        @pl.when(i0 + 2 < NCHUNKS)
        def _():
            gather_start(i0 + 2, 0)

        gather_wait(i0 + 1, 1)

        @pl.when(i0 >= 2)
        def _():
            wb_wait(i0 - 1, 1)

        repack(1)
        wb_start(i0 + 1, 1)
        return carry

    lax.fori_loop(0, NCHUNKS // 2, loop_body, 0)
    wb_wait(NCHUNKS - 2, 0)
    wb_wait(NCHUNKS - 1, 1)


def kernel(sentence, table):
    idx = sentence.reshape(NW, NCHUNKS, CHUNK)
    table_p = jnp.pad(table, ((0, 0), (0, EMB_PAD - EMB_DIM)))
    return _gather_kernel(table_p, idx)
